# baseline (speedup 1.0000x reference)
import functools

import jax
import jax.numpy as jnp
from jax import lax
from jax.experimental import pallas as pl
from jax.experimental.pallas import tpu as pltpu
from jax.experimental.pallas import tpu_sc as plsc

B = 8
NGT = 50
DN = 100
TOTAL = B * DN * NGT
PERB = DN * NGT
HID = 256

NW = 32
WPB = 4
QPW = 1280
QPB = WPB * QPW
QPAD = B * QPB
CHUNK = 128
NCH = QPW // CHUNK
BPW = 5120
BPB = WPB * BPW
BPAD = B * BPB
LDBL = 72
BDBL = 400

@functools.cache
def _build_sc():
    mesh = plsc.VectorSubcoreMesh(core_axis_name="c", subcore_axis_name="s")
    return pl.kernel(
        _sc_body,
        mesh=mesh,
        out_type=(
            jax.ShapeDtypeStruct((QPAD, HID), jnp.float32),
            jax.ShapeDtypeStruct((QPAD,), jnp.int32),
            jax.ShapeDtypeStruct((BPAD,), jnp.float32),
            jax.ShapeDtypeStruct((BPAD,), jnp.float32),
        ),
        scratch_types=[
            pltpu.VMEM((LDBL,), jnp.int32),
            pltpu.VMEM((BDBL,), jnp.float32),
            pltpu.VMEM((BDBL,), jnp.float32),
            pltpu.VMEM((QPW,), jnp.float32),
            pltpu.VMEM((QPW,), jnp.int32),
            pltpu.VMEM((NCH, CHUNK), jnp.int32),
            pltpu.VMEM((QPW,), jnp.int32),
            pltpu.VMEM((CHUNK, HID), jnp.float32),
            pltpu.VMEM((BPW,), jnp.float32),
            pltpu.VMEM((BPW,), jnp.float32),
            pltpu.VMEM((BPW,), jnp.float32),
            pltpu.SemaphoreType.DMA,
        ],
    )


def _sc_body(ldbl_h, bdbl_h, bcomp_h, nu_h, rl_h, bnr_h, table_h,
             q_out, lab_out, ref_out, tbox_out,
             ldbl_v, bdbl_v, bcomp_v, nu_v, rl_v, idx_v, lab_v, rows_v,
             bnr_v, refo_v, tbo_v, sem):
    wid = lax.axis_index("s") * 2 + lax.axis_index("c")
    b = wid // WPB
    lw = lax.rem(wid, WPB)
    qbase = b * QPB + lw * QPW
    bbase = b * BPB + lw * BPW
    qphase0 = lw * QPW
    bphase0 = lw * BPW

    pltpu.sync_copy(ldbl_h.at[pl.ds(b * LDBL, LDBL)], ldbl_v)
    pltpu.sync_copy(bdbl_h.at[pl.ds(b * BDBL, BDBL)], bdbl_v)
    pltpu.sync_copy(bcomp_h.at[pl.ds(b * BDBL, BDBL)], bcomp_v)
    pltpu.sync_copy(nu_h.at[pl.ds(qbase, QPW)], nu_v)
    pltpu.sync_copy(rl_h.at[pl.ds(qbase, QPW)], rl_v)
    pltpu.sync_copy(bnr_h.at[pl.ds(bbase, BPW)], bnr_v)

    for j in range(NCH):
        def grp(k, carry, j=j):
            off = j * CHUNK + k * 16
            phase = lax.rem(qphase0 + off, NGT)
            rep = ldbl_v[pl.ds(phase, 16)]
            lab_v[pl.ds(off, 16)] = rep
            nu = nu_v[pl.ds(off, 16)]
            rl = rl_v[pl.ds(off, 16)]
            idx_v[j, pl.ds(k * 16, 16)] = jnp.where(nu < 0.5, rl, rep)
            return carry
        lax.fori_loop(0, CHUNK // 16, grp, 0)
        pltpu.async_copy(table_h.at[idx_v.at[j]], rows_v, sem).wait()
        pltpu.sync_copy(rows_v, q_out.at[pl.ds(qbase + j * CHUNK, CHUNK)])

    def bgrp(i, carry):
        off = i * 16
        bphase = lax.rem(bphase0 + off, 4 * NGT)
        rb = bdbl_v[pl.ds(bphase, 16)]
        cb = bcomp_v[pl.ds(bphase, 16)]
        bn = bnr_v[pl.ds(off, 16)] * 0.8 - 0.4
        out = jnp.minimum(jnp.maximum(rb + bn * cb, 0.0), 1.0)
        refo_v[pl.ds(off, 16)] = out
        tbo_v[pl.ds(off, 16)] = rb
        return carry
    lax.fori_loop(0, BPW // 16, bgrp, 0)

    pltpu.sync_copy(lab_v, lab_out.at[pl.ds(qbase, QPW)])
    pltpu.sync_copy(refo_v, ref_out.at[pl.ds(bbase, BPW)])
    pltpu.sync_copy(tbo_v, tbox_out.at[pl.ds(bbase, BPW)])


def kernel(labels, boxes, noise_u, rand_labels, box_noise_raw, table):
    labels = labels.astype(jnp.int32)
    ldbl = jnp.concatenate([labels, labels[:, : LDBL - NGT]], axis=1)
    boxes_r = boxes.reshape(B, 4 * NGT)
    bdbl = jnp.concatenate([boxes_r, boxes_r[:, : BDBL - 4 * NGT]], axis=1)
    comp_r = boxes[:, :, (2, 3, 2, 3)].reshape(B, 4 * NGT)
    bcomp = jnp.concatenate([comp_r, comp_r[:, : BDBL - 4 * NGT]], axis=1)
    nu = jnp.pad(noise_u, ((0, 0), (0, QPB - PERB)))
    rl = jnp.pad(rand_labels.astype(jnp.int32), ((0, 0), (0, QPB - PERB)))
    bnr = jnp.pad(box_noise_raw.reshape(B, 4 * PERB), ((0, 0), (0, BPB - 4 * PERB)))
    q, lab, refo, tbo = _build_sc()(
        ldbl.reshape(-1), bdbl.reshape(-1), bcomp.reshape(-1),
        nu.reshape(-1), rl.reshape(-1), bnr.reshape(-1), table)
    dn_query = q.reshape(B, QPB, HID)[:, :PERB]
    dn_ref = refo.reshape(B, BPB)[:, : 4 * PERB].reshape(B, PERB, 4)
    dn_query_pos = jnp.zeros_like(dn_query)
    dn_target_labels = lab.reshape(B, QPB)[:, :PERB]
    dn_target_boxes = tbo.reshape(B, BPB)[:, : 4 * PERB].reshape(B, PERB, 4)
    return (dn_query, dn_ref, dn_query_pos, dn_target_labels, dn_target_boxes)
